# baseline (speedup 1.0000x reference)
import jax
import jax.numpy as jnp
from jax.experimental import pallas as pl
from jax.experimental.pallas import tpu as pltpu

_FEAT_STRIDE = 32
_POOL = 7
_C = 8
_H = 1024
_W = 1024
_N = 8192
_B = 128
_U = 16
_RB = 128


def _hilo_dot(a, b, dims):
    ah = a.astype(jnp.bfloat16).astype(jnp.float32)
    al = a - ah
    hi = jax.lax.dot_general(ah, b, dimension_numbers=dims,
                             preferred_element_type=jnp.float32)
    lo = jax.lax.dot_general(al, b, dimension_numbers=dims,
                             preferred_element_type=jnp.float32)
    return hi + lo


def _hilo_dot_l(a, b, dims):
    bh = b.astype(jnp.bfloat16).astype(jnp.float32)
    bl = b - bh
    hi = jax.lax.dot_general(a, bh, dimension_numbers=dims,
                             preferred_element_type=jnp.float32)
    lo = jax.lax.dot_general(a, bl, dimension_numbers=dims,
                             preferred_element_type=jnp.float32)
    return hi + lo


def _sat_kernel(x_ref, triu_ref, tril_ref, out_ref, carry_ref):
    yb = pl.program_id(0)

    @pl.when(yb == 0)
    def _():
        carry_ref[...] = jnp.zeros_like(carry_ref)

    for c in range(_C):
        X = x_ref[c]
        Yt = _hilo_dot(X, triu_ref[...], (((2,), (0,)), ((), ())))
        tot = jnp.sum(X, axis=2, keepdims=True)
        s = tot
        for sh in (1, 2, 4):
            zpad = jnp.zeros((_RB, sh, 1), jnp.float32)
            s = s + jnp.concatenate([zpad, s[:, :8 - sh, :]], axis=1)
        Xc = Yt + (s - tot)
        Yc = _hilo_dot_l(tril_ref[...], Xc,
                         (((1,), (0,)), ((), ())))
        Yc = Yc + carry_ref[c][None, :, :]
        carry_ref[c] = Yc[_RB - 1]
        out_ref[c::_C] = Yc


def _pool_kernel(s2_ref, yidx_ref, ywt_ref, w0_ref, xw_ref, cls_ref,
                 out_ref):
    lane = jax.lax.broadcasted_iota(jnp.int32, (_C, _B), 1)

    def arm(wtiles):
        wl = wtiles * 128

        def chunk(ci, acc):
            for u in range(_U):
                i = ci * _U + u
                wo = pl.multiple_of(w0_ref[0, 0, i] * 128, 128)
                i0 = pl.multiple_of(yidx_ref[i, 0], _C)
                g = ywt_ref[i, 0] * s2_ref[pl.ds(i0, _C), pl.ds(wo, wl)]
                for k in range(1, 2 * _POOL):
                    ik = pl.multiple_of(yidx_ref[i, k], _C)
                    g = g + ywt_ref[i, k] * s2_ref[pl.ds(ik, _C),
                                                   pl.ds(wo, wl)]
                xwt = pltpu.roll(xw_ref[i], -w0_ref[0, 0, i], axis=0)
                s = None
                for t in range(wtiles):
                    xb = jnp.broadcast_to(xwt[t:t + 1, :], (_C, 128))
                    p = g[:, t * 128:(t + 1) * 128] * xb
                    s = p if s is None else s + p
                res = jnp.sum(s, axis=1, keepdims=True)
                acc = jnp.where(lane == i, res, acc)
            return acc

        acc = jax.lax.fori_loop(0, _B // _U, chunk,
                                jnp.zeros((_C, _B), jnp.float32))
        out_ref[0] = acc

    bcls = cls_ref[0, 0, 0]
    for cl, wtiles in enumerate((2, 4, 6, 8)):
        @pl.when(bcls == cl)
        def _(wtiles=wtiles):
            arm(wtiles)


def _adaptive_terms(start, end):
    j = jnp.arange(_POOL)
    length = end - start
    lo = start[:, None] + (j[None, :] * length[:, None]) // _POOL
    hi = start[:, None] + ((j[None, :] + 1) * length[:, None]
                           + (_POOL - 1)) // _POOL
    d = (hi - lo).astype(jnp.float32)
    idx = jnp.concatenate([hi, lo], axis=1)
    wt = jnp.concatenate([1.0 / d, -1.0 / d], axis=1)
    wt = jnp.where(idx == 0, 0.0, wt)
    idx = jnp.maximum(idx - 1, 0)
    return idx.astype(jnp.int32), wt


def kernel(conv_out, rois):
    xv = conv_out[0].reshape(_C, _H, _W // 128, 128)
    triu = (jnp.arange(128)[:, None] <= jnp.arange(128)[None, :]
            ).astype(jnp.float32)
    s2i = pl.pallas_call(
        _sat_kernel,
        grid=(_H // _RB,),
        in_specs=[
            pl.BlockSpec((_C, _RB, _W // 128, 128),
                         lambda yb: (0, yb, 0, 0)),
            pl.BlockSpec(memory_space=pltpu.VMEM),
            pl.BlockSpec(memory_space=pltpu.VMEM),
        ],
        out_specs=pl.BlockSpec((_RB * _C, _W // 128, 128),
                               lambda yb: (yb, 0, 0)),
        out_shape=jax.ShapeDtypeStruct((_H * _C, _W // 128, 128),
                                       jnp.float32),
        scratch_shapes=[pltpu.VMEM((_C, _W // 128, 128), jnp.float32)],
        compiler_params=pltpu.CompilerParams(
            dimension_semantics=("arbitrary",),
            vmem_limit_bytes=56 * 1024 * 1024,
        ),
        name="sat_prep",
    )(xv, triu, triu.T)
    s2 = s2i.reshape(_H * _C, _W)

    r = (rois // _FEAT_STRIDE).astype(jnp.int32)
    xidx0, _xw0 = _adaptive_terms(r[:, 1], r[:, 3] + 1)
    t0 = jnp.min(xidx0, axis=1) // 128
    t1 = jnp.max(xidx0, axis=1) // 128
    wt_ = ((t1 - t0 + 2) // 2) * 2
    cls = wt_ // 2 - 1
    perm = jnp.argsort(cls)
    r = r[perm]

    yidx, ywt = _adaptive_terms(r[:, 0], r[:, 2] + 1)
    xidx, xwt = _adaptive_terms(r[:, 1], r[:, 3] + 1)
    ywt = ywt / float(_POOL * _POOL)
    yidx = yidx * _C

    blk_cls = jnp.max(cls[perm].reshape(_N // _B, _B),
                      axis=1, keepdims=True).astype(jnp.int32)
    wtb = (blk_cls + 1) * 2
    w0 = jnp.minimum((jnp.min(xidx, axis=1) // 128).reshape(_N // _B, _B),
                     (_W // 128) - wtb).astype(jnp.int32)

    xcol = jnp.arange(_W, dtype=jnp.int32)[None, None, :]
    xw_dense = jnp.sum(
        jnp.where(xcol == xidx[:, :, None], xwt[:, :, None], 0.0), axis=1)
    xw3 = xw_dense.reshape(_N, _W // 128, 128)

    out = pl.pallas_call(
        _pool_kernel,
        grid=(_N // _B,),
        in_specs=[
            pl.BlockSpec(memory_space=pltpu.VMEM),
            pl.BlockSpec((_B, 2 * _POOL), lambda i: (i, 0),
                         memory_space=pltpu.SMEM),
            pl.BlockSpec((_B, 2 * _POOL), lambda i: (i, 0),
                         memory_space=pltpu.SMEM),
            pl.BlockSpec((1, 1, _B), lambda i: (i, 0, 0),
                         memory_space=pltpu.SMEM),
            pl.BlockSpec((_B, _W // 128, 128), lambda i: (i, 0, 0)),
            pl.BlockSpec((1, 1, 1), lambda i: (i, 0, 0),
                         memory_space=pltpu.SMEM),
        ],
        out_specs=pl.BlockSpec((1, _C, _B), lambda i: (i, 0, 0)),
        out_shape=jax.ShapeDtypeStruct((_N // _B, _C, _B), jnp.float32),
        compiler_params=pltpu.CompilerParams(
            dimension_semantics=("arbitrary",),
            vmem_limit_bytes=56 * 1024 * 1024,
        ),
        name="roi_pool_sat",
    )(s2, yidx, ywt, w0.reshape(_N // _B, 1, _B),
      xw3, blk_cls.reshape(_N // _B, 1, 1))

    pooled = out.transpose(0, 2, 1).reshape(_N, _C)
    pooled = pooled[jnp.argsort(perm)]
    return pooled.reshape(-1, 2, 4)


# speedup vs baseline: 1.0141x; 1.0141x over previous
import jax
import jax.numpy as jnp
from jax.experimental import pallas as pl
from jax.experimental.pallas import tpu as pltpu

_FEAT_STRIDE = 32
_POOL = 7
_C = 8
_H = 1024
_W = 1024
_N = 8192
_B = 128
_U = 8
_RB = 128


def _hilo_dot(a, b, dims):
    ah = a.astype(jnp.bfloat16).astype(jnp.float32)
    al = a - ah
    hi = jax.lax.dot_general(ah, b, dimension_numbers=dims,
                             preferred_element_type=jnp.float32)
    lo = jax.lax.dot_general(al, b, dimension_numbers=dims,
                             preferred_element_type=jnp.float32)
    return hi + lo


def _hilo_dot_l(a, b, dims):
    bh = b.astype(jnp.bfloat16).astype(jnp.float32)
    bl = b - bh
    hi = jax.lax.dot_general(a, bh, dimension_numbers=dims,
                             preferred_element_type=jnp.float32)
    lo = jax.lax.dot_general(a, bl, dimension_numbers=dims,
                             preferred_element_type=jnp.float32)
    return hi + lo


def _sat_kernel(x_ref, triu_ref, tril_ref, out_ref, carry_ref):
    yb = pl.program_id(0)

    @pl.when(yb == 0)
    def _():
        carry_ref[...] = jnp.zeros_like(carry_ref)

    for c in range(_C):
        X = x_ref[c]
        Yt = _hilo_dot(X, triu_ref[...], (((2,), (0,)), ((), ())))
        tot = jnp.sum(X, axis=2, keepdims=True)
        s = tot
        for sh in (1, 2, 4):
            zpad = jnp.zeros((_RB, sh, 1), jnp.float32)
            s = s + jnp.concatenate([zpad, s[:, :8 - sh, :]], axis=1)
        Xc = Yt + (s - tot)
        Yc = _hilo_dot_l(tril_ref[...], Xc,
                         (((1,), (0,)), ((), ())))
        Yc = Yc + carry_ref[c][None, :, :]
        carry_ref[c] = Yc[_RB - 1]
        out_ref[c::_C] = Yc


def _pool_kernel(s2_ref, yidx_ref, ywt_ref, w0_ref, xw_ref, cls_ref,
                 out_ref):
    lane = jax.lax.broadcasted_iota(jnp.int32, (_C, _B), 1)

    def arm(wtiles):
        wl = wtiles * 128

        def chunk(ci, acc):
            for u in range(_U):
                i = ci * _U + u
                wo = pl.multiple_of(w0_ref[0, 0, i] * 128, 128)
                i0 = pl.multiple_of(yidx_ref[i, 0], _C)
                g = ywt_ref[i, 0] * s2_ref[pl.ds(i0, _C), pl.ds(wo, wl)]
                for k in range(1, 2 * _POOL):
                    ik = pl.multiple_of(yidx_ref[i, k], _C)
                    g = g + ywt_ref[i, k] * s2_ref[pl.ds(ik, _C),
                                                   pl.ds(wo, wl)]
                xwt = pltpu.roll(xw_ref[i], -w0_ref[0, 0, i], axis=0)
                s = None
                for t in range(wtiles):
                    xb = jnp.broadcast_to(xwt[t:t + 1, :], (_C, 128))
                    p = g[:, t * 128:(t + 1) * 128] * xb
                    s = p if s is None else s + p
                res = jnp.sum(s, axis=1, keepdims=True)
                acc = jnp.where(lane == i, res, acc)
            return acc

        acc = jax.lax.fori_loop(0, _B // _U, chunk,
                                jnp.zeros((_C, _B), jnp.float32))
        out_ref[0] = acc

    bcls = cls_ref[0, 0, 0]
    for cl, wtiles in enumerate((2, 4, 6, 8)):
        @pl.when(bcls == cl)
        def _(wtiles=wtiles):
            arm(wtiles)


def _adaptive_terms(start, end):
    j = jnp.arange(_POOL)
    length = end - start
    lo = start[:, None] + (j[None, :] * length[:, None]) // _POOL
    hi = start[:, None] + ((j[None, :] + 1) * length[:, None]
                           + (_POOL - 1)) // _POOL
    d = (hi - lo).astype(jnp.float32)
    idx = jnp.concatenate([hi, lo], axis=1)
    wt = jnp.concatenate([1.0 / d, -1.0 / d], axis=1)
    wt = jnp.where(idx == 0, 0.0, wt)
    idx = jnp.maximum(idx - 1, 0)
    return idx.astype(jnp.int32), wt


def kernel(conv_out, rois):
    xv = conv_out[0].reshape(_C, _H, _W // 128, 128)
    triu = (jnp.arange(128)[:, None] <= jnp.arange(128)[None, :]
            ).astype(jnp.float32)
    s2i = pl.pallas_call(
        _sat_kernel,
        grid=(_H // _RB,),
        in_specs=[
            pl.BlockSpec((_C, _RB, _W // 128, 128),
                         lambda yb: (0, yb, 0, 0)),
            pl.BlockSpec(memory_space=pltpu.VMEM),
            pl.BlockSpec(memory_space=pltpu.VMEM),
        ],
        out_specs=pl.BlockSpec((_RB * _C, _W // 128, 128),
                               lambda yb: (yb, 0, 0)),
        out_shape=jax.ShapeDtypeStruct((_H * _C, _W // 128, 128),
                                       jnp.float32),
        scratch_shapes=[pltpu.VMEM((_C, _W // 128, 128), jnp.float32)],
        compiler_params=pltpu.CompilerParams(
            dimension_semantics=("arbitrary",),
            vmem_limit_bytes=56 * 1024 * 1024,
        ),
        name="sat_prep",
    )(xv, triu, triu.T)
    s2 = s2i.reshape(_H * _C, _W)

    r = (rois // _FEAT_STRIDE).astype(jnp.int32)
    yidx, ywt = _adaptive_terms(r[:, 0], r[:, 2] + 1)
    xidx, xwt = _adaptive_terms(r[:, 1], r[:, 3] + 1)
    ywt = ywt / float(_POOL * _POOL)
    yidx = yidx * _C

    t0 = jnp.min(xidx, axis=1) // 128
    t1 = jnp.max(xidx, axis=1) // 128
    wt_ = ((t1 - t0 + 2) // 2) * 2
    cls = wt_ // 2 - 1

    perm = jnp.argsort(cls)
    yidx, ywt = yidx[perm], ywt[perm]
    xidx, xwt = xidx[perm], xwt[perm]
    blk_cls = jnp.max(cls[perm].reshape(_N // _B, _B),
                      axis=1, keepdims=True).astype(jnp.int32)
    wtb = (blk_cls + 1) * 2
    w0 = jnp.minimum(t0[perm].reshape(_N // _B, _B),
                     (_W // 128) - wtb).astype(jnp.int32)

    xcol = jnp.arange(_W, dtype=jnp.int32)[None, None, :]
    xw_dense = jnp.sum(
        jnp.where(xcol == xidx[:, :, None], xwt[:, :, None], 0.0), axis=1)
    xw3 = xw_dense.reshape(_N, _W // 128, 128)

    out = pl.pallas_call(
        _pool_kernel,
        grid=(_N // _B,),
        in_specs=[
            pl.BlockSpec(memory_space=pltpu.VMEM),
            pl.BlockSpec((_B, 2 * _POOL), lambda i: (i, 0),
                         memory_space=pltpu.SMEM),
            pl.BlockSpec((_B, 2 * _POOL), lambda i: (i, 0),
                         memory_space=pltpu.SMEM),
            pl.BlockSpec((1, 1, _B), lambda i: (i, 0, 0),
                         memory_space=pltpu.SMEM),
            pl.BlockSpec((_B, _W // 128, 128), lambda i: (i, 0, 0)),
            pl.BlockSpec((1, 1, 1), lambda i: (i, 0, 0),
                         memory_space=pltpu.SMEM),
        ],
        out_specs=pl.BlockSpec((1, _C, _B), lambda i: (i, 0, 0)),
        out_shape=jax.ShapeDtypeStruct((_N // _B, _C, _B), jnp.float32),
        compiler_params=pltpu.CompilerParams(
            dimension_semantics=("arbitrary",),
            vmem_limit_bytes=56 * 1024 * 1024,
        ),
        name="roi_pool_sat",
    )(s2, yidx, ywt, w0.reshape(_N // _B, 1, _B),
      xw3, blk_cls.reshape(_N // _B, 1, 1))

    pooled = out.transpose(0, 2, 1).reshape(_N, _C)
    pooled = pooled[jnp.argsort(perm)]
    return pooled.reshape(-1, 2, 4)
